# Initial kernel scaffold; baseline (speedup 1.0000x reference)
import jax
import jax.numpy as jnp
from jax.experimental import pallas as pl


def kernel(hidden_states, attention_mask, position_cos, position_sin, token_types, start_indices, end_indices, input_ln_w, post_ln_w, q_w, q_b, k_w, k_b, v_w, v_b, o_w, gate_w, up_w, down_w):
    raise NotImplementedError("write your pallas kernel here")


# speedup vs baseline: 1.6651x; 1.6651x over previous
import functools

import jax
import jax.numpy as jnp
from jax import lax
from jax.experimental import pallas as pl
from jax.experimental.pallas import tpu as pltpu
from jax.experimental.pallas import tpu_sc as plsc

B, S, H = 1, 2048, 2048
NH, NKV, DH = 16, 4, 128
E, I = 8, 2048
EPS = 1e-6

BS_M = 128
NB = S // BS_M
NWI = NB + E - 1
IC = 512
NIC = I // IC

ROWS_S = 256
NRB = S // ROWS_S

SC_NC, SC_NS = 2, 16
SC_NW = SC_NC * SC_NS
ROWS_W = S // SC_NW
CHUNK = 16
NCHUNK = ROWS_W // CHUNK


def _qkv_body(x_ref, lnw_ref, qw_ref, kw_ref, vw_ref, qb_ref, kb_ref, vb_ref,
              q_out, k_out, v_out):
    x = x_ref[...]
    var = jnp.mean(x * x, axis=-1, keepdims=True)
    xn = (x * lax.rsqrt(var + EPS)) * lnw_ref[...]
    dn = (((1,), (1,)), ((), ()))
    q_out[...] = lax.dot_general(xn, qw_ref[...], dn,
                                 preferred_element_type=jnp.float32) + qb_ref[...]
    k_out[...] = lax.dot_general(xn, kw_ref[...], dn,
                                 preferred_element_type=jnp.float32) + kb_ref[...]
    v_out[...] = lax.dot_general(xn, vw_ref[...], dn,
                                 preferred_element_type=jnp.float32) + vb_ref[...]


def _qkv_call(x, lnw, q_w, k_w, v_w, q_b, k_b, v_b):
    return pl.pallas_call(
        _qkv_body,
        grid=(NRB,),
        in_specs=[
            pl.BlockSpec((ROWS_S, H), lambda i: (i, 0)),
            pl.BlockSpec((1, H), lambda i: (0, 0)),
            pl.BlockSpec((NH * DH, H), lambda i: (0, 0)),
            pl.BlockSpec((NKV * DH, H), lambda i: (0, 0)),
            pl.BlockSpec((NKV * DH, H), lambda i: (0, 0)),
            pl.BlockSpec((1, NH * DH), lambda i: (0, 0)),
            pl.BlockSpec((1, NKV * DH), lambda i: (0, 0)),
            pl.BlockSpec((1, NKV * DH), lambda i: (0, 0)),
        ],
        out_specs=[
            pl.BlockSpec((ROWS_S, NH * DH), lambda i: (i, 0)),
            pl.BlockSpec((ROWS_S, NKV * DH), lambda i: (i, 0)),
            pl.BlockSpec((ROWS_S, NKV * DH), lambda i: (i, 0)),
        ],
        out_shape=[
            jax.ShapeDtypeStruct((S, NH * DH), jnp.float32),
            jax.ShapeDtypeStruct((S, NKV * DH), jnp.float32),
            jax.ShapeDtypeStruct((S, NKV * DH), jnp.float32),
        ],
        compiler_params=pltpu.CompilerParams(
            dimension_semantics=("arbitrary",)),
    )(x, lnw, q_w, k_w, v_w, q_b, k_b, v_b)


SQ = 512
NSQ = S // SQ


def _rope(x, cos, sin):
    half = DH // 2
    x1 = x[:, :half]
    x2 = x[:, half:]
    rot = jnp.concatenate([-x2, x1], axis=1)
    return x * cos + rot * sin


def _attn_body(q_ref, k_ref, v_ref, cq_ref, sq_ref, ck_ref, sk_ref, out_ref):
    sq_i = pl.program_id(1)
    q = _rope(q_ref[...], cq_ref[...], sq_ref[...])
    k = _rope(k_ref[...], ck_ref[...], sk_ref[...])
    scores = lax.dot_general(q, k, (((1,), (1,)), ((), ())),
                             preferred_element_type=jnp.float32)
    scores = scores * (1.0 / (DH ** 0.5))
    rows = lax.broadcasted_iota(jnp.int32, (SQ, S), 0) + sq_i * SQ
    cols = lax.broadcasted_iota(jnp.int32, (SQ, S), 1)
    scores = jnp.where(cols <= rows, scores, -1e9)
    m = jnp.max(scores, axis=-1, keepdims=True)
    p = jnp.exp(scores - m)
    p = p / jnp.sum(p, axis=-1, keepdims=True)
    out_ref[...] = lax.dot_general(p, v_ref[...], (((1,), (0,)), ((), ())),
                                   preferred_element_type=jnp.float32)


def _attn_call(q, k, v, cos2d, sin2d):
    return pl.pallas_call(
        _attn_body,
        grid=(NH, NSQ),
        in_specs=[
            pl.BlockSpec((SQ, DH), lambda h, sq: (sq, h)),
            pl.BlockSpec((S, DH), lambda h, sq: (0, h // (NH // NKV))),
            pl.BlockSpec((S, DH), lambda h, sq: (0, h // (NH // NKV))),
            pl.BlockSpec((SQ, DH), lambda h, sq: (sq, 0)),
            pl.BlockSpec((SQ, DH), lambda h, sq: (sq, 0)),
            pl.BlockSpec((S, DH), lambda h, sq: (0, 0)),
            pl.BlockSpec((S, DH), lambda h, sq: (0, 0)),
        ],
        out_specs=pl.BlockSpec((SQ, DH), lambda h, sq: (sq, h)),
        out_shape=jax.ShapeDtypeStruct((S, NH * DH), jnp.float32),
        compiler_params=pltpu.CompilerParams(
            dimension_semantics=("arbitrary", "arbitrary")),
    )(q, k, v, cos2d, sin2d, cos2d, sin2d)


def _oproj_body(a_ref, ow_ref, hs_ref, plnw_ref, hid_out, x2_out):
    h = hs_ref[...] + lax.dot_general(
        a_ref[...], ow_ref[...], (((1,), (1,)), ((), ())),
        preferred_element_type=jnp.float32)
    hid_out[...] = h
    var = jnp.mean(h * h, axis=-1, keepdims=True)
    x2_out[...] = (h * lax.rsqrt(var + EPS)) * plnw_ref[...]


def _oproj_call(attn_out, o_w, hs, plnw):
    return pl.pallas_call(
        _oproj_body,
        grid=(NRB,),
        in_specs=[
            pl.BlockSpec((ROWS_S, NH * DH), lambda i: (i, 0)),
            pl.BlockSpec((H, NH * DH), lambda i: (0, 0)),
            pl.BlockSpec((ROWS_S, H), lambda i: (i, 0)),
            pl.BlockSpec((1, H), lambda i: (0, 0)),
        ],
        out_specs=[
            pl.BlockSpec((ROWS_S, H), lambda i: (i, 0)),
            pl.BlockSpec((ROWS_S, H), lambda i: (i, 0)),
        ],
        out_shape=[
            jax.ShapeDtypeStruct((S, H), jnp.float32),
            jax.ShapeDtypeStruct((S, H), jnp.float32),
        ],
        compiler_params=pltpu.CompilerParams(
            dimension_semantics=("arbitrary",)),
    )(attn_out, o_w, hs, plnw)


def _route_body(tt_ref, start_ref, dest_ref):
    t = tt_ref[...]
    e_col = lax.broadcasted_iota(jnp.int32, (E, S), 0)
    oh = (jnp.broadcast_to(t, (E, S)) == e_col).astype(jnp.float32)
    ri = lax.broadcasted_iota(jnp.int32, (S, S), 0)
    ci = lax.broadcasted_iota(jnp.int32, (S, S), 1)
    tri = (ri <= ci).astype(jnp.float32)
    rank_incl = lax.dot_general(oh, tri, (((1,), (0,)), ((), ())),
                                preferred_element_type=jnp.float32)
    dest = jnp.zeros((1, S), jnp.int32)
    for e in range(E):
        r_e = rank_incl[e:e + 1, :].astype(jnp.int32)
        dest = jnp.where(t == e, start_ref[e] + r_e - 1, dest)
    dest_ref[...] = jnp.broadcast_to(dest, (8, S))


def _route_call(token_types2d, start_indices):
    return pl.pallas_call(
        _route_body,
        grid=(1,),
        in_specs=[
            pl.BlockSpec((1, S), lambda i: (0, 0)),
            pl.BlockSpec(memory_space=pltpu.SMEM),
        ],
        out_specs=pl.BlockSpec((8, S), lambda i: (0, 0)),
        out_shape=jax.ShapeDtypeStruct((8, S), jnp.int32),
    )(token_types2d, start_indices)


def _sc_mesh():
    return plsc.VectorSubcoreMesh(core_axis_name="c", subcore_axis_name="s")


def _sc_permute2(x, hid, dest2d):
    @functools.partial(
        pl.kernel, mesh=_sc_mesh(),
        out_type=[jax.ShapeDtypeStruct((S, H), jnp.float32),
                  jax.ShapeDtypeStruct((S, H), jnp.float32)],
        scratch_types=[pltpu.VMEM((NCHUNK, CHUNK), jnp.int32),
                       pltpu.VMEM((CHUNK, H), jnp.float32),
                       pltpu.SemaphoreType.DMA],
    )
    def kfn(x_hbm, hid_hbm, dest_hbm, xp_hbm, hp_hbm, idx_v, buf, sem):
        wid = lax.axis_index("s") * SC_NC + lax.axis_index("c")
        base = wid * ROWS_W
        pltpu.sync_copy(dest_hbm.at[pl.ds(wid * NCHUNK, NCHUNK)], idx_v)
        for j in range(NCHUNK):
            pltpu.sync_copy(x_hbm.at[pl.ds(base + j * CHUNK, CHUNK)], buf)
            pltpu.async_copy(buf, xp_hbm.at[idx_v.at[j]], sem).wait()
            pltpu.sync_copy(hid_hbm.at[pl.ds(base + j * CHUNK, CHUNK)], buf)
            pltpu.async_copy(buf, hp_hbm.at[idx_v.at[j]], sem).wait()

    return kfn(x, hid, dest2d)


def _sc_gather(yp, dest2d):
    @functools.partial(
        pl.kernel, mesh=_sc_mesh(),
        out_type=jax.ShapeDtypeStruct((S, H), jnp.float32),
        scratch_types=[pltpu.VMEM((NCHUNK, CHUNK), jnp.int32),
                       pltpu.VMEM((CHUNK, H), jnp.float32),
                       pltpu.SemaphoreType.DMA],
    )
    def kfn(yp_hbm, dest_hbm, out_hbm, idx_v, buf, sem):
        wid = lax.axis_index("s") * SC_NC + lax.axis_index("c")
        base = wid * ROWS_W
        pltpu.sync_copy(dest_hbm.at[pl.ds(wid * NCHUNK, NCHUNK)], idx_v)
        for j in range(NCHUNK):
            pltpu.async_copy(yp_hbm.at[idx_v.at[j]], buf, sem).wait()
            pltpu.sync_copy(buf, out_hbm.at[pl.ds(base + j * CHUNK, CHUNK)])

    return kfn(yp, dest2d)


def _build_table(start, end):
    b_ids = jnp.arange(NB, dtype=jnp.int32)
    lo = jnp.maximum(start[:, None], b_ids[None, :] * BS_M)
    hi = jnp.minimum(end[:, None], (b_ids[None, :] + 1) * BS_M)
    e_mat = jnp.broadcast_to(jnp.arange(E, dtype=jnp.int32)[:, None], (E, NB))
    b_mat = jnp.broadcast_to(b_ids[None, :], (E, NB))
    valid = lo < hi
    key = jnp.where(valid, e_mat * NB + b_mat, jnp.int32(E * NB))
    order = jnp.argsort(key.reshape(-1))
    fe = e_mat.reshape(-1)[order][:NWI]
    fb = b_mat.reshape(-1)[order][:NWI]
    flo = lo.reshape(-1)[order][:NWI].astype(jnp.int32)
    fhi = hi.reshape(-1)[order][:NWI].astype(jnp.int32)
    fv = key.reshape(-1)[order][:NWI] < E * NB
    nv = jnp.sum(valid.astype(jnp.int32))
    e_pad = fe[nv - 1]
    b_pad = fb[nv - 1]
    fe = jnp.where(fv, fe, e_pad)
    fb = jnp.where(fv, fb, b_pad)
    flo = jnp.where(fv, flo, 0)
    fhi = jnp.where(fv, fhi, 0)
    prev_b = jnp.concatenate([jnp.full((1,), -1, jnp.int32), fb[:-1]])
    first = jnp.logical_and(fv, fb != prev_b).astype(jnp.int32)
    return jnp.stack([fe, fb, flo, fhi, first])


def _moe_body(tbl_ref, x_ref, hp_ref, gw_ref, uw_ref, dw_ref, out_ref):
    wi = pl.program_id(0)
    ic = pl.program_id(1)
    lo = tbl_ref[2, wi]
    hi = tbl_ref[3, wi]
    first = tbl_ref[4, wi]
    base = tbl_ref[1, wi] * BS_M

    @pl.when(jnp.logical_and(first == 1, ic == 0))
    def _():
        out_ref[...] = hp_ref[...]

    @pl.when(hi > lo)
    def _():
        x = x_ref[...]
        dn = (((1,), (0,)), ((), ()))
        g = lax.dot_general(x, gw_ref[0], dn,
                            preferred_element_type=jnp.float32)
        u = lax.dot_general(x, uw_ref[0], dn,
                            preferred_element_type=jnp.float32)
        hmid = g * jax.nn.sigmoid(g) * u
        rows = lax.broadcasted_iota(jnp.int32, (BS_M, 1), 0) + base
        mask = jnp.logical_and(rows >= lo, rows < hi).astype(jnp.float32)
        out_ref[...] += lax.dot_general(hmid * mask, dw_ref[0], dn,
                                        preferred_element_type=jnp.float32)


def _moe_call(tbl, xp, hp, gate_w, up_w, down_w):
    def ic_eff(wi, ic, t):
        return jnp.where(t[2, wi] < t[3, wi], ic, NIC - 1)

    grid_spec = pltpu.PrefetchScalarGridSpec(
        num_scalar_prefetch=1,
        grid=(NWI, NIC),
        in_specs=[
            pl.BlockSpec((BS_M, H), lambda wi, ic, t: (t[1, wi], 0)),
            pl.BlockSpec((BS_M, H), lambda wi, ic, t: (t[1, wi], 0)),
            pl.BlockSpec((1, H, IC), lambda wi, ic, t: (t[0, wi], 0, ic_eff(wi, ic, t))),
            pl.BlockSpec((1, H, IC), lambda wi, ic, t: (t[0, wi], 0, ic_eff(wi, ic, t))),
            pl.BlockSpec((1, IC, H), lambda wi, ic, t: (t[0, wi], ic_eff(wi, ic, t), 0)),
        ],
        out_specs=pl.BlockSpec((BS_M, H), lambda wi, ic, t: (t[1, wi], 0)),
    )
    return pl.pallas_call(
        _moe_body,
        grid_spec=grid_spec,
        out_shape=jax.ShapeDtypeStruct((S, H), jnp.float32),
        compiler_params=pltpu.CompilerParams(
            dimension_semantics=("arbitrary", "arbitrary")),
    )(tbl, xp, hp, gate_w, up_w, down_w)


def kernel(hidden_states, attention_mask, position_cos, position_sin,
           token_types, start_indices, end_indices, input_ln_w, post_ln_w,
           q_w, q_b, k_w, k_b, v_w, v_b, o_w, gate_w, up_w, down_w):
    x = hidden_states.reshape(S, H)
    cos2d = position_cos[0, 0]
    sin2d = position_sin[0, 0]

    q, k, v = _qkv_call(x, input_ln_w.reshape(1, H), q_w, k_w, v_w,
                        q_b.reshape(1, NH * DH), k_b.reshape(1, NKV * DH),
                        v_b.reshape(1, NKV * DH))
    attn_out = _attn_call(q, k, v, cos2d, sin2d)
    hid, x2 = _oproj_call(attn_out, o_w, x, post_ln_w.reshape(1, H))

    dest = _route_call(token_types.reshape(1, S), start_indices)[0]
    dest2d = dest.reshape(S // CHUNK, CHUNK)

    xp, hp = _sc_permute2(x2, hid, dest2d)
    tbl = _build_table(start_indices, end_indices)
    yp = _moe_call(tbl, xp, hp, gate_w, up_w, down_w)
    out = _sc_gather(yp, dest2d)
    return out.reshape(B, S, H)
